# Initial kernel scaffold; baseline (speedup 1.0000x reference)
import jax
import jax.numpy as jnp
from jax.experimental import pallas as pl


def kernel(edge_index, edge_weight, home, away, emb, W1, b1, W2, b2, W3, b3, L1W, L1b, L2W, L2b, L3W, L3b):
    raise NotImplementedError("write your pallas kernel here")


# speedup vs baseline: 6.7636x; 6.7636x over previous
import functools

import jax
import jax.numpy as jnp
from jax import lax
from jax.experimental import pallas as pl
from jax.experimental.pallas import tpu as pltpu
from jax.experimental.pallas import tpu_sc as plsc

N_NODES = 10000
EMBED = 128
HID = 128
DENSE = 128
TARGET = 3
BATCH = 4096
N_EDGES = 320000

NC = 2
NS = 16
NW = NC * NS
CHUNK = 128
NE_PAD = ((N_EDGES + NW * CHUNK - 1) // (NW * CHUNK)) * (NW * CHUNK)
EPW = NE_PAD // NW
NCHUNKS = EPW // CHUNK
NPAD = 10240
NROWS = 10240
ROWS_PER_TILE = NROWS // NS

_MESH = plsc.VectorSubcoreMesh(core_axis_name="c", subcore_axis_name="s")


@functools.partial(
    pl.kernel,
    out_type=jax.ShapeDtypeStruct((NC, NPAD), jnp.float32),
    mesh=_MESH,
    scratch_types=[
        pltpu.VMEM((CHUNK,), jnp.int32),
        pltpu.VMEM((CHUNK,), jnp.float32),
        pltpu.VMEM_SHARED((NPAD,), jnp.float32),
    ],
)
def _deg_sc(dst_hbm, ew_hbm, zero1_hbm, out_hbm, dst_v, ew_v, acc):
    c = lax.axis_index("c")
    s = lax.axis_index("s")
    wid = c * NS + s
    pltpu.sync_copy(zero1_hbm.at[pl.ds(s * 640, 640)], acc.at[pl.ds(s * 640, 640)])
    plsc.subcore_barrier()

    def body(i, carry):
        base = wid * EPW + i * CHUNK
        pltpu.sync_copy(dst_hbm.at[pl.ds(base, CHUNK)], dst_v)
        pltpu.sync_copy(ew_hbm.at[pl.ds(base, CHUNK)], ew_v)
        pltpu.sync_copy(ew_v, acc.at[dst_v], add=True)
        return carry

    lax.fori_loop(0, NCHUNKS, body, 0)
    plsc.subcore_barrier()

    @pl.when(s == 0)
    def _():
        pltpu.sync_copy(acc, out_hbm.at[c])


@functools.partial(
    pl.kernel,
    out_type=jax.ShapeDtypeStruct((NC, NROWS, HID), jnp.float32),
    mesh=_MESH,
    scratch_types=[
        pltpu.VMEM((CHUNK,), jnp.int32),
        pltpu.VMEM((CHUNK,), jnp.int32),
        pltpu.VMEM((CHUNK,), jnp.float32),
        pltpu.VMEM((CHUNK, HID), jnp.float32),
        pltpu.VMEM_SHARED((NROWS, HID), jnp.float32),
        pltpu.SemaphoreType.DMA,
    ],
)
def _agg_sc(y_hbm, src_hbm, dst_hbm, ew_hbm, zero2_hbm, out_hbm,
            src_v, dst_v, ew_v, rows_v, acc, sem):
    c = lax.axis_index("c")
    s = lax.axis_index("s")
    wid = c * NS + s
    pltpu.sync_copy(zero2_hbm.at[pl.ds(s * ROWS_PER_TILE, ROWS_PER_TILE), :],
                    acc.at[pl.ds(s * ROWS_PER_TILE, ROWS_PER_TILE), :])
    plsc.subcore_barrier()

    def chunk_body(i, carry):
        base = wid * EPW + i * CHUNK
        pltpu.sync_copy(src_hbm.at[pl.ds(base, CHUNK)], src_v)
        pltpu.sync_copy(dst_hbm.at[pl.ds(base, CHUNK)], dst_v)
        pltpu.sync_copy(ew_hbm.at[pl.ds(base, CHUNK)], ew_v)
        pltpu.async_copy(y_hbm.at[src_v], rows_v, sem).wait()

        def scale_body(g, carry2):
            wv = ew_v[pl.ds(g * 16, 16)]
            for j in range(16):
                w = lax.gather(
                    wv, jnp.full((16, 1), j, jnp.int32),
                    dimension_numbers=lax.GatherDimensionNumbers(
                        offset_dims=(), collapsed_slice_dims=(0,),
                        start_index_map=(0,)),
                    slice_sizes=(1,),
                    mode=lax.GatherScatterMode.PROMISE_IN_BOUNDS)
                e = g * 16 + j
                for d in range(HID // 16):
                    rows_v[e, pl.ds(d * 16, 16)] = rows_v[e, pl.ds(d * 16, 16)] * w
            return carry2

        lax.fori_loop(0, CHUNK // 16, scale_body, 0)
        pltpu.sync_copy(rows_v, acc.at[dst_v], add=True)
        return carry

    lax.fori_loop(0, NCHUNKS, chunk_body, 0)
    plsc.subcore_barrier()
    pltpu.sync_copy(acc.at[pl.ds(s * ROWS_PER_TILE, ROWS_PER_TILE), :],
                    out_hbm.at[c, pl.ds(s * ROWS_PER_TILE, ROWS_PER_TILE), :])


@functools.partial(
    pl.kernel,
    out_type=jax.ShapeDtypeStruct((2 * BATCH, HID), jnp.float32),
    mesh=_MESH,
    scratch_types=[
        pltpu.VMEM((CHUNK,), jnp.int32),
        pltpu.VMEM((CHUNK, HID), jnp.float32),
        pltpu.SemaphoreType.DMA,
    ],
)
def _gather_sc(x_hbm, idx_hbm, out_hbm, idx_v, rows_v, sem):
    c = lax.axis_index("c")
    s = lax.axis_index("s")
    wid = c * NS + s
    per_w = (2 * BATCH) // NW
    for j in range(per_w // CHUNK):
        base = wid * per_w + j * CHUNK
        pltpu.sync_copy(idx_hbm.at[pl.ds(base, CHUNK)], idx_v)
        pltpu.async_copy(x_hbm.at[idx_v], rows_v, sem).wait()
        pltpu.sync_copy(rows_v, out_hbm.at[pl.ds(base, CHUNK), :])


_BLK = 1000


def _tc_first_body(degp_ref, emb_ref, w_ref, y_ref, dinv_ref):
    deg = jnp.sum(degp_ref[...], axis=1) + 1.0
    dinv = lax.rsqrt(deg)[:, None]
    xw = jnp.dot(emb_ref[...], w_ref[...], preferred_element_type=jnp.float32)
    y_ref[...] = dinv * xw
    dinv_ref[...] = dinv


def _tc_mid_body(agg_ref, y_ref, dinv_ref, b_ref, w_ref, ynext_ref):
    a = agg_ref[0] + agg_ref[1] + y_ref[...]
    x = jnp.maximum(dinv_ref[...] * a + b_ref[...], 0.0)
    ynext_ref[...] = dinv_ref[...] * jnp.dot(
        x, w_ref[...], preferred_element_type=jnp.float32)


def _tc_last_body(agg_ref, y_ref, dinv_ref, b_ref, x_ref):
    a = agg_ref[0] + agg_ref[1] + y_ref[...]
    x_ref[...] = jnp.maximum(dinv_ref[...] * a + b_ref[...], 0.0)


def _tc_mlp_body(gh_ref, ga_ref, w1t_ref, w1b_ref, b1_ref, w2_ref, b2_ref,
                 w3_ref, b3_ref, out_ref):
    h = jnp.dot(gh_ref[...], w1t_ref[...], preferred_element_type=jnp.float32)
    h = h + jnp.dot(ga_ref[...], w1b_ref[...], preferred_element_type=jnp.float32)
    h = jnp.maximum(h + b1_ref[...], 0.0)
    h = jnp.maximum(
        jnp.dot(h, w2_ref[...], preferred_element_type=jnp.float32) + b2_ref[...], 0.0)
    logit = jnp.maximum(
        jnp.dot(h, w3_ref[...], preferred_element_type=jnp.float32) + b3_ref[...], 0.0)
    col = lax.broadcasted_iota(jnp.int32, logit.shape, 1)
    valid = col < TARGET
    lm = jnp.where(valid, logit, -1e30)
    m = jnp.max(lm, axis=1, keepdims=True)
    ssum = jnp.sum(jnp.where(valid, jnp.exp(lm - m), 0.0), axis=1, keepdims=True)
    out_ref[...] = logit - m - jnp.log(ssum)


def _row_specs(nrows, blk, *shapes_full):
    return [pl.BlockSpec((blk,) + s, lambda i: (i,) + (0,) * len(s))
            for s in shapes_full]


def kernel(edge_index, edge_weight, home, away, emb, W1, b1, W2, b2, W3, b3,
           L1W, L1b, L2W, L2b, L3W, L3b):
    f32 = jnp.float32
    src = edge_index[0].astype(jnp.int32)
    dst = edge_index[1].astype(jnp.int32)
    ew = edge_weight.astype(f32)
    pad = NE_PAD - N_EDGES
    src_p = jnp.concatenate([src, jnp.zeros((pad,), jnp.int32)])
    dst_p = jnp.concatenate([dst, jnp.zeros((pad,), jnp.int32)])
    ew_p = jnp.concatenate([ew, jnp.zeros((pad,), f32)])
    zero1 = jnp.zeros((NPAD,), f32)
    zero2 = jnp.zeros((NROWS, HID), f32)
    idx_all = jnp.concatenate([home, away]).astype(jnp.int32)

    degp = _deg_sc(dst_p, ew_p, zero1)

    grid = (N_NODES // _BLK,)
    y1, dinv = pl.pallas_call(
        _tc_first_body,
        grid=grid,
        in_specs=[
            pl.BlockSpec((_BLK, NC), lambda i: (i, 0)),
            pl.BlockSpec((_BLK, EMBED), lambda i: (i, 0)),
            pl.BlockSpec((EMBED, HID), lambda i: (0, 0)),
        ],
        out_specs=[
            pl.BlockSpec((_BLK, HID), lambda i: (i, 0)),
            pl.BlockSpec((_BLK, 1), lambda i: (i, 0)),
        ],
        out_shape=[
            jax.ShapeDtypeStruct((N_NODES, HID), f32),
            jax.ShapeDtypeStruct((N_NODES, 1), f32),
        ],
    )(degp[:, :N_NODES].T, emb, W1)

    def mid_layer(y, b, w_next):
        aggp = _agg_sc(y, src_p, dst_p, ew_p, zero2)
        return pl.pallas_call(
            _tc_mid_body,
            grid=grid,
            in_specs=[
                pl.BlockSpec((NC, _BLK, HID), lambda i: (0, i, 0)),
                pl.BlockSpec((_BLK, HID), lambda i: (i, 0)),
                pl.BlockSpec((_BLK, 1), lambda i: (i, 0)),
                pl.BlockSpec((1, HID), lambda i: (0, 0)),
                pl.BlockSpec((HID, HID), lambda i: (0, 0)),
            ],
            out_specs=pl.BlockSpec((_BLK, HID), lambda i: (i, 0)),
            out_shape=jax.ShapeDtypeStruct((N_NODES, HID), f32),
        )(aggp, y, dinv, b.reshape(1, HID), w_next)

    y2 = mid_layer(y1, b1, W2)
    y3 = mid_layer(y2, b2, W3)
    aggp3 = _agg_sc(y3, src_p, dst_p, ew_p, zero2)
    x3 = pl.pallas_call(
        _tc_last_body,
        grid=grid,
        in_specs=[
            pl.BlockSpec((NC, _BLK, HID), lambda i: (0, i, 0)),
            pl.BlockSpec((_BLK, HID), lambda i: (i, 0)),
            pl.BlockSpec((_BLK, 1), lambda i: (i, 0)),
            pl.BlockSpec((1, HID), lambda i: (0, 0)),
        ],
        out_specs=pl.BlockSpec((_BLK, HID), lambda i: (i, 0)),
        out_shape=jax.ShapeDtypeStruct((N_NODES, HID), f32),
    )(aggp3, y3, dinv, b3.reshape(1, HID))

    gathered = _gather_sc(x3, idx_all)
    gh = gathered[:BATCH]
    ga = gathered[BATCH:]

    w3p = jnp.zeros((DENSE, 128), f32).at[:, :TARGET].set(L3W)
    b3p = jnp.zeros((1, 128), f32).at[0, :TARGET].set(L3b)
    mblk = 512
    mgrid = (BATCH // mblk,)
    full = pl.pallas_call(
        _tc_mlp_body,
        grid=mgrid,
        in_specs=[
            pl.BlockSpec((mblk, HID), lambda i: (i, 0)),
            pl.BlockSpec((mblk, HID), lambda i: (i, 0)),
            pl.BlockSpec((HID, DENSE), lambda i: (0, 0)),
            pl.BlockSpec((HID, DENSE), lambda i: (0, 0)),
            pl.BlockSpec((1, DENSE), lambda i: (0, 0)),
            pl.BlockSpec((DENSE, DENSE), lambda i: (0, 0)),
            pl.BlockSpec((1, DENSE), lambda i: (0, 0)),
            pl.BlockSpec((DENSE, 128), lambda i: (0, 0)),
            pl.BlockSpec((1, 128), lambda i: (0, 0)),
        ],
        out_specs=pl.BlockSpec((mblk, 128), lambda i: (i, 0)),
        out_shape=jax.ShapeDtypeStruct((BATCH, 128), f32),
    )(gh, ga, L1W[:HID], L1W[HID:], L1b.reshape(1, DENSE), L2W,
      L2b.reshape(1, DENSE), w3p, b3p)
    return full[:, :TARGET]


# speedup vs baseline: 7.8584x; 1.1619x over previous
import functools

import jax
import jax.numpy as jnp
from jax import lax
from jax.experimental import pallas as pl
from jax.experimental.pallas import tpu as pltpu
from jax.experimental.pallas import tpu_sc as plsc

N_NODES = 10000
EMBED = 128
HID = 128
DENSE = 128
TARGET = 3
BATCH = 4096
N_EDGES = 320000

NC = 2
NS = 16
NW = NC * NS
CHUNK = 128
NCHUNKS = 80
EPW = NCHUNKS * CHUNK
NE_PAD = NW * EPW
NPAD = 10240
NROWS = 10240
ROWS_PER_TILE = NROWS // NS

_MESH = plsc.VectorSubcoreMesh(core_axis_name="c", subcore_axis_name="s")


@functools.partial(
    pl.kernel,
    out_type=jax.ShapeDtypeStruct((NC, NPAD), jnp.float32),
    mesh=_MESH,
    scratch_types=[
        pltpu.VMEM((NCHUNKS, CHUNK), jnp.int32),
        pltpu.VMEM((NCHUNKS, CHUNK), jnp.float32),
        pltpu.VMEM_SHARED((NPAD,), jnp.float32),
        pltpu.SemaphoreType.DMA,
    ],
)
def _deg_sc(dst3_hbm, ew3_hbm, zero1_hbm, out_hbm, dsts_v, ews_v, acc, sem):
    c = lax.axis_index("c")
    s = lax.axis_index("s")
    wid = c * NS + s
    pltpu.sync_copy(dst3_hbm.at[wid], dsts_v)
    pltpu.sync_copy(ew3_hbm.at[wid], ews_v)
    pltpu.sync_copy(zero1_hbm.at[pl.ds(s * 640, 640)], acc.at[pl.ds(s * 640, 640)])
    plsc.subcore_barrier()

    G = 8

    def body(i4, carry):
        descs = [
            pltpu.async_copy(ews_v.at[i4 * G + j], acc.at[dsts_v.at[i4 * G + j]],
                             sem, add=True)
            for j in range(G)
        ]
        for dsc in descs:
            dsc.wait()
        return carry

    lax.fori_loop(0, NCHUNKS // G, body, 0)
    plsc.subcore_barrier()

    @pl.when(s == 0)
    def _():
        pltpu.sync_copy(acc, out_hbm.at[c])


@functools.partial(
    pl.kernel,
    out_type=jax.ShapeDtypeStruct((NC, NROWS, HID), jnp.float32),
    mesh=_MESH,
    scratch_types=[
        pltpu.VMEM((NCHUNKS // 2, CHUNK), jnp.int32),
        pltpu.VMEM((NCHUNKS // 2, CHUNK), jnp.int32),
        pltpu.VMEM((NCHUNKS // 2, CHUNK), jnp.float32),
        pltpu.VMEM((CHUNK, HID), jnp.float32),
        pltpu.VMEM((CHUNK, HID), jnp.float32),
        pltpu.VMEM_SHARED((NROWS, HID), jnp.float32),
        pltpu.SemaphoreType.DMA,
        pltpu.SemaphoreType.DMA,
    ],
)
def _agg_sc(y_hbm, src3_hbm, dst3_hbm, ew3_hbm, zero2_hbm, out_hbm,
            srcs_v, dsts_v, ews_v, rows0, rows1, acc, gsem0, gsem1):
    c = lax.axis_index("c")
    s = lax.axis_index("s")
    wid = c * NS + s
    HALF = NCHUNKS // 2
    pltpu.sync_copy(zero2_hbm.at[pl.ds(s * ROWS_PER_TILE, ROWS_PER_TILE), :],
                    acc.at[pl.ds(s * ROWS_PER_TILE, ROWS_PER_TILE), :])
    plsc.subcore_barrier()

    def scale(rows, i):
        def scale_body(g, carry2):
            wv = ews_v[i, pl.ds(g * 16, 16)]
            for j in range(16):
                w = lax.gather(
                    wv, jnp.full((16, 1), j, jnp.int32),
                    dimension_numbers=lax.GatherDimensionNumbers(
                        offset_dims=(), collapsed_slice_dims=(0,),
                        start_index_map=(0,)),
                    slice_sizes=(1,),
                    mode=lax.GatherScatterMode.PROMISE_IN_BOUNDS)
                e = g * 16 + j
                for d in range(HID // 16):
                    rows[e, pl.ds(d * 16, 16)] = rows[e, pl.ds(d * 16, 16)] * w
            return carry2

        lax.fori_loop(0, CHUNK // 16, scale_body, 0)

    def wait_rows(sem, rows):
        pltpu.make_async_copy(y_hbm.at[srcs_v.at[0]], rows, sem).wait()

    def half_body(h, carry):
        pltpu.sync_copy(src3_hbm.at[wid, pl.ds(h * HALF, HALF)], srcs_v)
        pltpu.sync_copy(dst3_hbm.at[wid, pl.ds(h * HALF, HALF)], dsts_v)
        pltpu.sync_copy(ew3_hbm.at[wid, pl.ds(h * HALF, HALF)], ews_v)
        pltpu.async_copy(y_hbm.at[srcs_v.at[0]], rows0, gsem0)

        def body(i2, carry2):
            a = 2 * i2
            b = a + 1
            wait_rows(gsem0, rows0)
            pltpu.async_copy(y_hbm.at[srcs_v.at[b]], rows1, gsem1)
            scale(rows0, a)
            pltpu.sync_copy(rows0, acc.at[dsts_v.at[a]], add=True)
            wait_rows(gsem1, rows1)

            @pl.when(i2 + 1 < HALF // 2)
            def _():
                pltpu.async_copy(y_hbm.at[srcs_v.at[a + 2]], rows0, gsem0)

            scale(rows1, b)
            pltpu.sync_copy(rows1, acc.at[dsts_v.at[b]], add=True)
            return carry2

        lax.fori_loop(0, HALF // 2, body, 0)
        return carry

    lax.fori_loop(0, 2, half_body, 0)
    plsc.subcore_barrier()
    pltpu.sync_copy(acc.at[pl.ds(s * ROWS_PER_TILE, ROWS_PER_TILE), :],
                    out_hbm.at[c, pl.ds(s * ROWS_PER_TILE, ROWS_PER_TILE), :])


@functools.partial(
    pl.kernel,
    out_type=jax.ShapeDtypeStruct((2 * BATCH, HID), jnp.float32),
    mesh=_MESH,
    scratch_types=[
        pltpu.VMEM((CHUNK,), jnp.int32),
        pltpu.VMEM((CHUNK, HID), jnp.float32),
        pltpu.SemaphoreType.DMA,
    ],
)
def _gather_sc(x_hbm, idx_hbm, out_hbm, idx_v, rows_v, sem):
    c = lax.axis_index("c")
    s = lax.axis_index("s")
    wid = c * NS + s
    per_w = (2 * BATCH) // NW
    for j in range(per_w // CHUNK):
        base = wid * per_w + j * CHUNK
        pltpu.sync_copy(idx_hbm.at[pl.ds(base, CHUNK)], idx_v)
        pltpu.async_copy(x_hbm.at[idx_v], rows_v, sem).wait()
        pltpu.sync_copy(rows_v, out_hbm.at[pl.ds(base, CHUNK), :])


_BLK = 1000


def _tc_first_body(degp_ref, emb_ref, w_ref, y_ref, dinv_ref):
    deg = jnp.sum(degp_ref[...], axis=1) + 1.0
    dinv = lax.rsqrt(deg)[:, None]
    xw = jnp.dot(emb_ref[...], w_ref[...], preferred_element_type=jnp.float32)
    y_ref[...] = dinv * xw
    dinv_ref[...] = dinv


def _tc_mid_body(agg_ref, y_ref, dinv_ref, b_ref, w_ref, ynext_ref):
    a = agg_ref[0] + agg_ref[1] + y_ref[...]
    x = jnp.maximum(dinv_ref[...] * a + b_ref[...], 0.0)
    ynext_ref[...] = dinv_ref[...] * jnp.dot(
        x, w_ref[...], preferred_element_type=jnp.float32)


def _tc_last_body(agg_ref, y_ref, dinv_ref, b_ref, x_ref):
    a = agg_ref[0] + agg_ref[1] + y_ref[...]
    x_ref[...] = jnp.maximum(dinv_ref[...] * a + b_ref[...], 0.0)


def _tc_mlp_body(gh_ref, ga_ref, w1t_ref, w1b_ref, b1_ref, w2_ref, b2_ref,
                 w3_ref, b3_ref, out_ref):
    h = jnp.dot(gh_ref[...], w1t_ref[...], preferred_element_type=jnp.float32)
    h = h + jnp.dot(ga_ref[...], w1b_ref[...], preferred_element_type=jnp.float32)
    h = jnp.maximum(h + b1_ref[...], 0.0)
    h = jnp.maximum(
        jnp.dot(h, w2_ref[...], preferred_element_type=jnp.float32) + b2_ref[...], 0.0)
    logit = jnp.maximum(
        jnp.dot(h, w3_ref[...], preferred_element_type=jnp.float32) + b3_ref[...], 0.0)
    col = lax.broadcasted_iota(jnp.int32, logit.shape, 1)
    valid = col < TARGET
    lm = jnp.where(valid, logit, -1e30)
    m = jnp.max(lm, axis=1, keepdims=True)
    ssum = jnp.sum(jnp.where(valid, jnp.exp(lm - m), 0.0), axis=1, keepdims=True)
    out_ref[...] = logit - m - jnp.log(ssum)


def _row_specs(nrows, blk, *shapes_full):
    return [pl.BlockSpec((blk,) + s, lambda i: (i,) + (0,) * len(s))
            for s in shapes_full]


def kernel(edge_index, edge_weight, home, away, emb, W1, b1, W2, b2, W3, b3,
           L1W, L1b, L2W, L2b, L3W, L3b):
    f32 = jnp.float32
    src = edge_index[0].astype(jnp.int32)
    dst = edge_index[1].astype(jnp.int32)
    ew = edge_weight.astype(f32)
    pad = NE_PAD - N_EDGES
    src_p = jnp.concatenate([src, jnp.zeros((pad,), jnp.int32)]).reshape(
        NW, NCHUNKS, CHUNK)
    dst_p = jnp.concatenate([dst, jnp.zeros((pad,), jnp.int32)]).reshape(
        NW, NCHUNKS, CHUNK)
    ew_p = jnp.concatenate([ew, jnp.zeros((pad,), f32)]).reshape(
        NW, NCHUNKS, CHUNK)
    zero1 = jnp.zeros((NPAD,), f32)
    zero2 = jnp.zeros((NROWS, HID), f32)
    idx_all = jnp.concatenate([home, away]).astype(jnp.int32)

    degp = _deg_sc(dst_p, ew_p, zero1)

    grid = (N_NODES // _BLK,)
    y1, dinv = pl.pallas_call(
        _tc_first_body,
        grid=grid,
        in_specs=[
            pl.BlockSpec((_BLK, NC), lambda i: (i, 0)),
            pl.BlockSpec((_BLK, EMBED), lambda i: (i, 0)),
            pl.BlockSpec((EMBED, HID), lambda i: (0, 0)),
        ],
        out_specs=[
            pl.BlockSpec((_BLK, HID), lambda i: (i, 0)),
            pl.BlockSpec((_BLK, 1), lambda i: (i, 0)),
        ],
        out_shape=[
            jax.ShapeDtypeStruct((N_NODES, HID), f32),
            jax.ShapeDtypeStruct((N_NODES, 1), f32),
        ],
    )(degp[:, :N_NODES].T, emb, W1)

    def mid_layer(y, b, w_next):
        aggp = _agg_sc(y, src_p, dst_p, ew_p, zero2)
        return pl.pallas_call(
            _tc_mid_body,
            grid=grid,
            in_specs=[
                pl.BlockSpec((NC, _BLK, HID), lambda i: (0, i, 0)),
                pl.BlockSpec((_BLK, HID), lambda i: (i, 0)),
                pl.BlockSpec((_BLK, 1), lambda i: (i, 0)),
                pl.BlockSpec((1, HID), lambda i: (0, 0)),
                pl.BlockSpec((HID, HID), lambda i: (0, 0)),
            ],
            out_specs=pl.BlockSpec((_BLK, HID), lambda i: (i, 0)),
            out_shape=jax.ShapeDtypeStruct((N_NODES, HID), f32),
        )(aggp, y, dinv, b.reshape(1, HID), w_next)

    y2 = mid_layer(y1, b1, W2)
    y3 = mid_layer(y2, b2, W3)
    aggp3 = _agg_sc(y3, src_p, dst_p, ew_p, zero2)
    x3 = pl.pallas_call(
        _tc_last_body,
        grid=grid,
        in_specs=[
            pl.BlockSpec((NC, _BLK, HID), lambda i: (0, i, 0)),
            pl.BlockSpec((_BLK, HID), lambda i: (i, 0)),
            pl.BlockSpec((_BLK, 1), lambda i: (i, 0)),
            pl.BlockSpec((1, HID), lambda i: (0, 0)),
        ],
        out_specs=pl.BlockSpec((_BLK, HID), lambda i: (i, 0)),
        out_shape=jax.ShapeDtypeStruct((N_NODES, HID), f32),
    )(aggp3, y3, dinv, b3.reshape(1, HID))

    gathered = _gather_sc(x3, idx_all)
    gh = gathered[:BATCH]
    ga = gathered[BATCH:]

    w3p = jnp.zeros((DENSE, 128), f32).at[:, :TARGET].set(L3W)
    b3p = jnp.zeros((1, 128), f32).at[0, :TARGET].set(L3b)
    mblk = 512
    mgrid = (BATCH // mblk,)
    full = pl.pallas_call(
        _tc_mlp_body,
        grid=mgrid,
        in_specs=[
            pl.BlockSpec((mblk, HID), lambda i: (i, 0)),
            pl.BlockSpec((mblk, HID), lambda i: (i, 0)),
            pl.BlockSpec((HID, DENSE), lambda i: (0, 0)),
            pl.BlockSpec((HID, DENSE), lambda i: (0, 0)),
            pl.BlockSpec((1, DENSE), lambda i: (0, 0)),
            pl.BlockSpec((DENSE, DENSE), lambda i: (0, 0)),
            pl.BlockSpec((1, DENSE), lambda i: (0, 0)),
            pl.BlockSpec((DENSE, 128), lambda i: (0, 0)),
            pl.BlockSpec((1, 128), lambda i: (0, 0)),
        ],
        out_specs=pl.BlockSpec((mblk, 128), lambda i: (i, 0)),
        out_shape=jax.ShapeDtypeStruct((BATCH, 128), f32),
    )(gh, ga, L1W[:HID], L1W[HID:], L1b.reshape(1, DENSE), L2W,
      L2b.reshape(1, DENSE), w3p, b3p)
    return full[:, :TARGET]


# speedup vs baseline: 21.8153x; 2.7760x over previous
import functools

import jax
import jax.numpy as jnp
from jax import lax
from jax.experimental import pallas as pl
from jax.experimental.pallas import tpu as pltpu
from jax.experimental.pallas import tpu_sc as plsc

N_NODES = 10000
EMBED = 128
HID = 128
DENSE = 128
TARGET = 3
BATCH = 4096
N_EDGES = 320000

NC = 2
NS = 16
NW = NC * NS
CHUNK = 128
NCHUNKS = 80
EPW = NCHUNKS * CHUNK
NE_PAD = NW * EPW
NPAD = 10240
NROWS = 10240
ROWS_PER_TILE = NROWS // NS

_MESH = plsc.VectorSubcoreMesh(core_axis_name="c", subcore_axis_name="s")


@functools.partial(
    pl.kernel,
    out_type=jax.ShapeDtypeStruct((NC, NPAD), jnp.float32),
    mesh=_MESH,
    scratch_types=[
        pltpu.VMEM((NCHUNKS, CHUNK), jnp.int32),
        pltpu.VMEM((NCHUNKS, CHUNK), jnp.float32),
        pltpu.VMEM_SHARED((NPAD,), jnp.float32),
        pltpu.SemaphoreType.DMA,
    ],
)
def _deg_sc(dst3_hbm, ew3_hbm, zero1_hbm, out_hbm, dsts_v, ews_v, acc, sem):
    c = lax.axis_index("c")
    s = lax.axis_index("s")
    wid = c * NS + s
    pltpu.sync_copy(dst3_hbm.at[wid], dsts_v)
    pltpu.sync_copy(ew3_hbm.at[wid], ews_v)
    pltpu.sync_copy(zero1_hbm.at[pl.ds(s * 640, 640)], acc.at[pl.ds(s * 640, 640)])
    plsc.subcore_barrier()

    G = 8

    def body(i4, carry):
        descs = [
            pltpu.async_copy(ews_v.at[i4 * G + j], acc.at[dsts_v.at[i4 * G + j]],
                             sem, add=True)
            for j in range(G)
        ]
        for dsc in descs:
            dsc.wait()
        return carry

    lax.fori_loop(0, NCHUNKS // G, body, 0)
    plsc.subcore_barrier()

    @pl.when(s == 0)
    def _():
        pltpu.sync_copy(acc, out_hbm.at[c])


@functools.partial(
    pl.kernel,
    out_type=jax.ShapeDtypeStruct((NC, NROWS, HID), jnp.float32),
    mesh=_MESH,
    scratch_types=[
        pltpu.VMEM((NCHUNKS // 2, CHUNK), jnp.int32),
        pltpu.VMEM((NCHUNKS // 2, CHUNK), jnp.int32),
        pltpu.VMEM((NCHUNKS // 2, CHUNK), jnp.float32),
        pltpu.VMEM((CHUNK, HID), jnp.float32),
        pltpu.VMEM((CHUNK, HID), jnp.float32),
        pltpu.VMEM_SHARED((NROWS, HID), jnp.float32),
        pltpu.SemaphoreType.DMA,
        pltpu.SemaphoreType.DMA,
    ],
)
def _agg_sc(y_hbm, src3_hbm, dst3_hbm, ew3_hbm, zero2_hbm, out_hbm,
            srcs_v, dsts_v, ews_v, rows0, rows1, acc, gsem0, gsem1):
    c = lax.axis_index("c")
    s = lax.axis_index("s")
    wid = c * NS + s
    HALF = NCHUNKS // 2
    pltpu.sync_copy(zero2_hbm.at[pl.ds(s * ROWS_PER_TILE, ROWS_PER_TILE), :],
                    acc.at[pl.ds(s * ROWS_PER_TILE, ROWS_PER_TILE), :])
    plsc.subcore_barrier()

    def scale(rows, i):
        def scale_body(g, carry2):
            wv = ews_v[i, pl.ds(g * 16, 16)]
            for j in range(16):
                w = lax.gather(
                    wv, jnp.full((16, 1), j, jnp.int32),
                    dimension_numbers=lax.GatherDimensionNumbers(
                        offset_dims=(), collapsed_slice_dims=(0,),
                        start_index_map=(0,)),
                    slice_sizes=(1,),
                    mode=lax.GatherScatterMode.PROMISE_IN_BOUNDS)
                e = g * 16 + j
                for d in range(HID // 16):
                    rows[e, pl.ds(d * 16, 16)] = rows[e, pl.ds(d * 16, 16)] * w
            return carry2

        lax.fori_loop(0, CHUNK // 16, scale_body, 0)

    def wait_rows(sem, rows):
        pltpu.make_async_copy(y_hbm.at[srcs_v.at[0]], rows, sem).wait()

    def half_body(h, carry):
        pltpu.sync_copy(src3_hbm.at[wid, pl.ds(h * HALF, HALF)], srcs_v)
        pltpu.sync_copy(dst3_hbm.at[wid, pl.ds(h * HALF, HALF)], dsts_v)
        pltpu.sync_copy(ew3_hbm.at[wid, pl.ds(h * HALF, HALF)], ews_v)
        pltpu.async_copy(y_hbm.at[srcs_v.at[0]], rows0, gsem0)

        def body(i2, carry2):
            a = 2 * i2
            b = a + 1
            wait_rows(gsem0, rows0)
            pltpu.async_copy(y_hbm.at[srcs_v.at[b]], rows1, gsem1)
            scale(rows0, a)
            pltpu.sync_copy(rows0, acc.at[dsts_v.at[a]], add=True)
            wait_rows(gsem1, rows1)

            @pl.when(i2 + 1 < HALF // 2)
            def _():
                pltpu.async_copy(y_hbm.at[srcs_v.at[a + 2]], rows0, gsem0)

            scale(rows1, b)
            pltpu.sync_copy(rows1, acc.at[dsts_v.at[b]], add=True)
            return carry2

        lax.fori_loop(0, HALF // 2, body, 0)
        return carry

    lax.fori_loop(0, 2, half_body, 0)
    plsc.subcore_barrier()
    pltpu.sync_copy(acc.at[pl.ds(s * ROWS_PER_TILE, ROWS_PER_TILE), :],
                    out_hbm.at[c, pl.ds(s * ROWS_PER_TILE, ROWS_PER_TILE), :])


@functools.partial(
    pl.kernel,
    out_type=jax.ShapeDtypeStruct((2 * BATCH, HID), jnp.float32),
    mesh=_MESH,
    scratch_types=[
        pltpu.VMEM((CHUNK,), jnp.int32),
        pltpu.VMEM((CHUNK, HID), jnp.float32),
        pltpu.SemaphoreType.DMA,
    ],
)
def _gather_sc(x_hbm, idx_hbm, out_hbm, idx_v, rows_v, sem):
    c = lax.axis_index("c")
    s = lax.axis_index("s")
    wid = c * NS + s
    per_w = (2 * BATCH) // NW
    for j in range(per_w // CHUNK):
        base = wid * per_w + j * CHUNK
        pltpu.sync_copy(idx_hbm.at[pl.ds(base, CHUNK)], idx_v)
        pltpu.async_copy(x_hbm.at[idx_v], rows_v, sem).wait()
        pltpu.sync_copy(rows_v, out_hbm.at[pl.ds(base, CHUNK), :])


_BLK = 1000


def _tc_first_body(degp_ref, emb_ref, w_ref, y_ref, dinv_ref):
    deg = jnp.sum(degp_ref[...], axis=1) + 1.0
    dinv = lax.rsqrt(deg)[:, None]
    xw = jnp.dot(emb_ref[...], w_ref[...], preferred_element_type=jnp.float32)
    y_ref[...] = dinv * xw
    dinv_ref[...] = dinv


def _tc_mid_body(agg_ref, y_ref, dinv_ref, b_ref, w_ref, ynext_ref):
    a = agg_ref[0] + agg_ref[1] + y_ref[...]
    x = jnp.maximum(dinv_ref[...] * a + b_ref[...], 0.0)
    ynext_ref[...] = dinv_ref[...] * jnp.dot(
        x, w_ref[...], preferred_element_type=jnp.float32)


def _tc_last_body(agg_ref, y_ref, dinv_ref, b_ref, x_ref):
    a = agg_ref[0] + agg_ref[1] + y_ref[...]
    x_ref[...] = jnp.maximum(dinv_ref[...] * a + b_ref[...], 0.0)


def _tc_mlp_body(gh_ref, ga_ref, w1t_ref, w1b_ref, b1_ref, w2_ref, b2_ref,
                 w3_ref, b3_ref, out_ref):
    h = jnp.dot(gh_ref[...], w1t_ref[...], preferred_element_type=jnp.float32)
    h = h + jnp.dot(ga_ref[...], w1b_ref[...], preferred_element_type=jnp.float32)
    h = jnp.maximum(h + b1_ref[...], 0.0)
    h = jnp.maximum(
        jnp.dot(h, w2_ref[...], preferred_element_type=jnp.float32) + b2_ref[...], 0.0)
    logit = jnp.maximum(
        jnp.dot(h, w3_ref[...], preferred_element_type=jnp.float32) + b3_ref[...], 0.0)
    col = lax.broadcasted_iota(jnp.int32, logit.shape, 1)
    valid = col < TARGET
    lm = jnp.where(valid, logit, -1e30)
    m = jnp.max(lm, axis=1, keepdims=True)
    ssum = jnp.sum(jnp.where(valid, jnp.exp(lm - m), 0.0), axis=1, keepdims=True)
    out_ref[...] = logit - m - jnp.log(ssum)


def _row_specs(nrows, blk, *shapes_full):
    return [pl.BlockSpec((blk,) + s, lambda i: (i,) + (0,) * len(s))
            for s in shapes_full]


def kernel(edge_index, edge_weight, home, away, emb, W1, b1, W2, b2, W3, b3,
           L1W, L1b, L2W, L2b, L3W, L3b):
    f32 = jnp.float32
    src = edge_index[0].astype(jnp.int32)
    dst = edge_index[1].astype(jnp.int32)
    ew = edge_weight.astype(f32)
    pad = NE_PAD - N_EDGES
    pad_idx = jnp.arange(pad, dtype=jnp.int32) % N_NODES
    src_p = jnp.concatenate([src, pad_idx]).reshape(NW, NCHUNKS, CHUNK)
    dst_p = jnp.concatenate([dst, pad_idx]).reshape(NW, NCHUNKS, CHUNK)
    ew_p = jnp.concatenate([ew, jnp.zeros((pad,), f32)]).reshape(
        NW, NCHUNKS, CHUNK)
    zero1 = jnp.zeros((NPAD,), f32)
    zero2 = jnp.zeros((NROWS, HID), f32)
    idx_all = jnp.concatenate([home, away]).astype(jnp.int32)

    degp = _deg_sc(dst_p, ew_p, zero1)

    grid = (N_NODES // _BLK,)
    y1, dinv = pl.pallas_call(
        _tc_first_body,
        grid=grid,
        in_specs=[
            pl.BlockSpec((_BLK, NC), lambda i: (i, 0)),
            pl.BlockSpec((_BLK, EMBED), lambda i: (i, 0)),
            pl.BlockSpec((EMBED, HID), lambda i: (0, 0)),
        ],
        out_specs=[
            pl.BlockSpec((_BLK, HID), lambda i: (i, 0)),
            pl.BlockSpec((_BLK, 1), lambda i: (i, 0)),
        ],
        out_shape=[
            jax.ShapeDtypeStruct((N_NODES, HID), f32),
            jax.ShapeDtypeStruct((N_NODES, 1), f32),
        ],
    )(degp[:, :N_NODES].T, emb, W1)

    def mid_layer(y, b, w_next):
        aggp = _agg_sc(y, src_p, dst_p, ew_p, zero2)
        return pl.pallas_call(
            _tc_mid_body,
            grid=grid,
            in_specs=[
                pl.BlockSpec((NC, _BLK, HID), lambda i: (0, i, 0)),
                pl.BlockSpec((_BLK, HID), lambda i: (i, 0)),
                pl.BlockSpec((_BLK, 1), lambda i: (i, 0)),
                pl.BlockSpec((1, HID), lambda i: (0, 0)),
                pl.BlockSpec((HID, HID), lambda i: (0, 0)),
            ],
            out_specs=pl.BlockSpec((_BLK, HID), lambda i: (i, 0)),
            out_shape=jax.ShapeDtypeStruct((N_NODES, HID), f32),
        )(aggp, y, dinv, b.reshape(1, HID), w_next)

    y2 = mid_layer(y1, b1, W2)
    y3 = mid_layer(y2, b2, W3)
    aggp3 = _agg_sc(y3, src_p, dst_p, ew_p, zero2)
    x3 = pl.pallas_call(
        _tc_last_body,
        grid=grid,
        in_specs=[
            pl.BlockSpec((NC, _BLK, HID), lambda i: (0, i, 0)),
            pl.BlockSpec((_BLK, HID), lambda i: (i, 0)),
            pl.BlockSpec((_BLK, 1), lambda i: (i, 0)),
            pl.BlockSpec((1, HID), lambda i: (0, 0)),
        ],
        out_specs=pl.BlockSpec((_BLK, HID), lambda i: (i, 0)),
        out_shape=jax.ShapeDtypeStruct((N_NODES, HID), f32),
    )(aggp3, y3, dinv, b3.reshape(1, HID))

    gathered = _gather_sc(x3, idx_all)
    gh = gathered[:BATCH]
    ga = gathered[BATCH:]

    w3p = jnp.zeros((DENSE, 128), f32).at[:, :TARGET].set(L3W)
    b3p = jnp.zeros((1, 128), f32).at[0, :TARGET].set(L3b)
    mblk = 512
    mgrid = (BATCH // mblk,)
    full = pl.pallas_call(
        _tc_mlp_body,
        grid=mgrid,
        in_specs=[
            pl.BlockSpec((mblk, HID), lambda i: (i, 0)),
            pl.BlockSpec((mblk, HID), lambda i: (i, 0)),
            pl.BlockSpec((HID, DENSE), lambda i: (0, 0)),
            pl.BlockSpec((HID, DENSE), lambda i: (0, 0)),
            pl.BlockSpec((1, DENSE), lambda i: (0, 0)),
            pl.BlockSpec((DENSE, DENSE), lambda i: (0, 0)),
            pl.BlockSpec((1, DENSE), lambda i: (0, 0)),
            pl.BlockSpec((DENSE, 128), lambda i: (0, 0)),
            pl.BlockSpec((1, 128), lambda i: (0, 0)),
        ],
        out_specs=pl.BlockSpec((mblk, 128), lambda i: (i, 0)),
        out_shape=jax.ShapeDtypeStruct((BATCH, 128), f32),
    )(gh, ga, L1W[:HID], L1W[HID:], L1b.reshape(1, DENSE), L2W,
      L2b.reshape(1, DENSE), w3p, b3p)
    return full[:, :TARGET]
